# baseline (speedup 1.0000x reference)
import functools

import jax
import jax.numpy as jnp
from jax import lax
from jax.experimental import pallas as pl
from jax.experimental.pallas import tpu as pltpu
from jax.experimental.pallas import tpu_sc as plsc

E = 8
NZ = 8
NEXP = E + NZ
TOPK = 2
D = 2048
DFF = 1408
T = 2048
SCALE = 2.5

BT = 256
BROW = 128


def _nblk():
    return (TOPK * T) // BROW + E


def _cap():
    return _nblk() * BROW


def _router_weights(x, wr, bias):
    logits = lax.dot_general(x, wr, (((1,), (1,)), ((), ())),
                             preferred_element_type=jnp.float32)
    scores = jax.nn.sigmoid(logits)
    biased = scores + bias
    ii = lax.broadcasted_iota(jnp.int32, biased.shape, 1)
    big = jnp.int32(NEXP)
    m1 = jnp.max(biased, axis=1, keepdims=True)
    i1 = jnp.min(jnp.where(biased >= m1, ii, big), axis=1, keepdims=True)
    oh1 = ii == i1
    b2 = jnp.where(oh1, -jnp.inf, biased)
    m2 = jnp.max(b2, axis=1, keepdims=True)
    i2 = jnp.min(jnp.where(b2 >= m2, ii, big), axis=1, keepdims=True)
    oh2 = ii == i2
    return jnp.where(oh1 | oh2, scores, 0.0)


def _router_body(x_ref, wr_ref, bias_ref, w_ref):
    w_ref[...] = _router_weights(x_ref[...], wr_ref[...], bias_ref[...])


def _router(x, wr, bias2d):
    return pl.pallas_call(
        _router_body,
        grid=(T // BT,),
        in_specs=[
            pl.BlockSpec((BT, D), lambda t: (t, 0)),
            pl.BlockSpec((NEXP, D), lambda t: (0, 0)),
            pl.BlockSpec((1, NEXP), lambda t: (0, 0)),
        ],
        out_specs=pl.BlockSpec((BT, NEXP), lambda t: (t, 0)),
        out_shape=jax.ShapeDtypeStruct((T, NEXP), jnp.float32),
    )(x, wr, bias2d)


def _dispatch_body(w_ref, posw_ref, meta_ref):
    w = w_ref[...]
    nslots = TOPK * T
    cap = _cap()
    nblk = _nblk()

    m = w > 0.0
    ii16 = lax.broadcasted_iota(jnp.int32, (T, NEXP), 1)
    e1 = jnp.min(jnp.where(m, ii16, NEXP + 1), axis=1, keepdims=True)
    e2 = jnp.max(jnp.where(m, ii16, -1), axis=1, keepdims=True)

    ii8 = lax.broadcasted_iota(jnp.int32, (T, E), 1)
    ind1 = ((ii8 == e1) & (e1 < E)).astype(jnp.float32)
    ind2 = ((ii8 == e2) & (e2 < E)).astype(jnp.float32)
    ind = jnp.concatenate([ind1, ind2], axis=0)

    chunk = 512 if nslots % 512 == 0 else nslots
    base = jnp.zeros((1, E), jnp.float32)
    r_chunks = []
    ir = lax.broadcasted_iota(jnp.int32, (chunk, chunk), 0)
    ic = lax.broadcasted_iota(jnp.int32, (chunk, chunk), 1)
    tri = (ic < ir).astype(jnp.float32)
    for c in range(nslots // chunk):
        ind_c = ind[c * chunk:(c + 1) * chunk]
        r_c = lax.dot_general(tri, ind_c, (((1,), (0,)), ((), ())),
                              preferred_element_type=jnp.float32)
        r_chunks.append(r_c + base)
        base = base + jnp.sum(ind_c, axis=0, keepdims=True)
    ranks = jnp.concatenate(r_chunks, axis=0)
    counts = base

    pc = jnp.floor((counts + (BROW - 1)) * (1.0 / BROW)) * BROW
    ie1 = lax.broadcasted_iota(jnp.int32, (E, E), 0)
    ie2 = lax.broadcasted_iota(jnp.int32, (E, E), 1)
    excl = (ie1 < ie2).astype(jnp.float32)
    starts = lax.dot_general(pc, excl, (((1,), (0,)), ((), ())),
                             preferred_element_type=jnp.float32)

    real = jnp.sum(ind, axis=1, keepdims=True) > 0.0
    pos_f = jnp.sum((ranks + starts) * ind, axis=1, keepdims=True)
    pos_f = jnp.where(real, pos_f, 0.0)

    slot_i = lax.broadcasted_iota(jnp.int32, (nslots, 1), 0).astype(jnp.float32)
    pos_s = jnp.where(real, pos_f, cap + slot_i)
    pos1 = pos_f[:T]
    pos2 = pos_f[T:]
    ps1 = pos_s[:T]
    ps2 = pos_s[T:]
    w1 = jnp.sum(w[:, :E] * ind1, axis=1, keepdims=True)
    w2 = jnp.sum(w[:, :E] * ind2, axis=1, keepdims=True)
    zero_w = jnp.sum(w[:, E:], axis=1, keepdims=True)
    pad = jnp.zeros((T, 128 - 7), jnp.float32)
    posw_ref[...] = jnp.concatenate(
        [pos1, pos2, w1, w2, zero_w, ps1, ps2, pad], axis=1)

    rowpos = (lax.broadcasted_iota(jnp.int32, (1, 128), 1) * BROW).astype(jnp.float32)
    be = jnp.zeros((1, 128), jnp.float32)
    for e in range(E):
        s_e = starts[:, e:e + 1]
        p_e = pc[:, e:e + 1]
        be = be + e * ((rowpos >= s_e) & (rowpos < s_e + p_e)).astype(jnp.float32)
    nused = jnp.sum(pc, axis=1, keepdims=True) * (1.0 / BROW)
    meta_ref[...] = jnp.concatenate([
        be.astype(jnp.int32),
        jnp.broadcast_to(nused.astype(jnp.int32), (1, 128)),
        jnp.zeros((6, 128), jnp.int32),
    ], axis=0)


def _dispatch(w_slots):
    return pl.pallas_call(
        _dispatch_body,
        grid=(1,),
        in_specs=[pl.BlockSpec((T, NEXP), lambda i: (0, 0))],
        out_specs=[
            pl.BlockSpec((T, 128), lambda i: (0, 0)),
            pl.BlockSpec((8, 128), lambda i: (0, 0)),
        ],
        out_shape=[
            jax.ShapeDtypeStruct((T, 128), jnp.float32),
            jax.ShapeDtypeStruct((8, 128), jnp.int32),
        ],
    )(w_slots)


def _sc_scatter_x(x, ps):
    n = ps.shape[0]
    wdt = x.shape[1]
    cap = _cap()
    info = plsc.get_sparse_core_info()
    nw = info.num_cores * info.num_subcores
    rows_w = n // nw
    mesh = plsc.VectorSubcoreMesh(core_axis_name="c", subcore_axis_name="s")

    chunk = rows_w
    while chunk * wdt * 4 > 260_000:
        chunk //= 2

    @functools.partial(
        pl.kernel, mesh=mesh,
        out_type=jax.ShapeDtypeStruct((cap + n, wdt), jnp.float32),
        scratch_types=[
            pltpu.VMEM((chunk,), jnp.int32),
            pltpu.VMEM((chunk, wdt), jnp.float32),
            pltpu.SemaphoreType.DMA,
        ],
    )
    def k(x_hbm, ps_hbm, xs_hbm, idx_v, rows_v, sem):
        wid = lax.axis_index("s") * info.num_cores + lax.axis_index("c")
        base = wid * rows_w
        for c in range(rows_w // chunk):
            tok0 = lax.rem(base + c * chunk, T)
            pltpu.sync_copy(ps_hbm.at[pl.ds(base + c * chunk, chunk)], idx_v)
            pltpu.sync_copy(x_hbm.at[pl.ds(tok0, chunk)], rows_v)
            pltpu.async_copy(rows_v, xs_hbm.at[idx_v], sem).wait()

    return k(x, ps)


def _sc_gather2(ys, idx1, idx2):
    n = idx1.shape[0]
    wdt = ys.shape[1]
    info = plsc.get_sparse_core_info()
    nw = info.num_cores * info.num_subcores
    rows_w = n // nw
    mesh = plsc.VectorSubcoreMesh(core_axis_name="c", subcore_axis_name="s")

    chunk = rows_w
    while chunk * wdt * 4 > 260_000:
        chunk //= 2

    @functools.partial(
        pl.kernel, mesh=mesh,
        out_type=(jax.ShapeDtypeStruct((n, wdt), jnp.int32),
                  jax.ShapeDtypeStruct((n, wdt), jnp.int32)),
        scratch_types=[
            pltpu.VMEM((chunk,), jnp.int32),
            pltpu.VMEM((chunk, wdt), jnp.int32),
            pltpu.SemaphoreType.DMA,
        ],
    )
    def k(ys_hbm, i1_hbm, i2_hbm, g1_hbm, g2_hbm, idx_v, rows_v, sem):
        wid = lax.axis_index("s") * info.num_cores + lax.axis_index("c")
        base = wid * rows_w
        for ih, oh in ((i1_hbm, g1_hbm), (i2_hbm, g2_hbm)):
            for c in range(rows_w // chunk):
                b = base + c * chunk
                pltpu.sync_copy(ih.at[pl.ds(b, chunk)], idx_v)
                pltpu.async_copy(ys_hbm.at[idx_v], rows_v, sem).wait()
                pltpu.sync_copy(rows_v, oh.at[pl.ds(b, chunk)])

    return k(ys, idx1, idx2)


def _ffn_body(be_ref, nu_ref, xs_ref, wg_ref, wu_ref, wd_ref, ys_ref):
    b = pl.program_id(0)

    @pl.when((b == 0) | (b < nu_ref[0]))
    def _go():
        x = xs_ref[...].astype(jnp.bfloat16)
        g = lax.dot_general(x, wg_ref[0], (((1,), (1,)), ((), ())),
                            preferred_element_type=jnp.float32)
        u = lax.dot_general(x, wu_ref[0], (((1,), (1,)), ((), ())),
                            preferred_element_type=jnp.float32)
        h = (g * jax.nn.sigmoid(g) * u).astype(jnp.bfloat16)
        y = lax.dot_general(h, wd_ref[0], (((1,), (1,)), ((), ())),
                            preferred_element_type=jnp.float32)
        ys_ref[...] = y.astype(jnp.bfloat16)


def _ffn(be, nu, xs, wgb, wub, wdb):
    cap = _cap()
    nblk = _nblk()
    grid_spec = pltpu.PrefetchScalarGridSpec(
        num_scalar_prefetch=2,
        grid=(nblk,),
        in_specs=[
            pl.BlockSpec((BROW, D), lambda b, be, nu: (jnp.minimum(b, jnp.maximum(nu[0] - 1, 0)), 0)),
            pl.BlockSpec((1, DFF, D), lambda b, be, nu: (be[b], 0, 0)),
            pl.BlockSpec((1, DFF, D), lambda b, be, nu: (be[b], 0, 0)),
            pl.BlockSpec((1, D, DFF), lambda b, be, nu: (be[b], 0, 0)),
        ],
        out_specs=pl.BlockSpec((BROW, D), lambda b, be, nu: (b, 0)),
    )
    return pl.pallas_call(
        _ffn_body,
        grid_spec=grid_spec,
        out_shape=jax.ShapeDtypeStruct((cap, D), jnp.bfloat16),
        compiler_params=pltpu.CompilerParams(
            dimension_semantics=("arbitrary",),
        ),
    )(be, nu, xs, wgb, wub, wdb)


def _combine_body(posw_ref, x_ref, g1_ref, g2_ref, out_ref):
    pw = posw_ref[...]
    w1 = pw[:, 2:3]
    w2 = pw[:, 3:4]
    zw = pw[:, 4:5]
    g1 = g1_ref[...].astype(jnp.float32)
    g2 = g2_ref[...].astype(jnp.float32)
    out_ref[...] = zw * x_ref[...] + SCALE * (w1 * g1 + w2 * g2)


def _combine(posw, x, g1, g2):
    return pl.pallas_call(
        _combine_body,
        grid=(T // BT,),
        in_specs=[
            pl.BlockSpec((BT, 128), lambda t: (t, 0)),
            pl.BlockSpec((BT, D), lambda t: (t, 0)),
            pl.BlockSpec((BT, D), lambda t: (t, 0)),
            pl.BlockSpec((BT, D), lambda t: (t, 0)),
        ],
        out_specs=pl.BlockSpec((BT, D), lambda t: (t, 0)),
        out_shape=jax.ShapeDtypeStruct((T, D), jnp.float32),
    )(posw, x, g1, g2)


@jax.jit
def _moe(x, wr, bias2d, wg, wu, wd):
    w_slots = _router(x, wr, bias2d)
    posw, meta = _dispatch(w_slots)
    be = meta[0, :_nblk()]
    nu = meta[1, :1]

    ps = jnp.concatenate([posw[:, 5], posw[:, 6]]).astype(jnp.int32)
    xs_full = _sc_scatter_x(x, ps)

    wgb = wg.astype(jnp.bfloat16)
    wub = wu.astype(jnp.bfloat16)
    wdb = wd.astype(jnp.bfloat16)
    ys = _ffn(be, nu, xs_full, wgb, wub, wdb)

    pos1 = posw[:, 0].astype(jnp.int32)
    pos2 = posw[:, 1].astype(jnp.int32)
    ys_i = lax.bitcast_convert_type(ys.reshape(_cap(), D // 2, 2), jnp.int32)
    g1_i, g2_i = _sc_gather2(ys_i, pos1, pos2)
    g1 = lax.bitcast_convert_type(g1_i, jnp.bfloat16).reshape(T, D)
    g2 = lax.bitcast_convert_type(g2_i, jnp.bfloat16).reshape(T, D)
    return _combine(posw, x, g1, g2)


def kernel(hidden_states, W_router, correction_bias, W_gate, W_up, W_down):
    bias2d = correction_bias.reshape(1, NEXP)
    return _moe(hidden_states, W_router, bias2d, W_gate, W_up, W_down)


# speedup vs baseline: 1.8822x; 1.8822x over previous
import functools

import jax
import jax.numpy as jnp
from jax import lax
from jax.experimental import pallas as pl
from jax.experimental.pallas import tpu as pltpu
from jax.experimental.pallas import tpu_sc as plsc

E = 8
NZ = 8
NEXP = E + NZ
TOPK = 2
D = 2048
DFF = 1408
T = 2048
SCALE = 2.5

BT = 256
BROW = 128


def _nblk():
    return (TOPK * T) // BROW + E


def _cap():
    return _nblk() * BROW


def _router_weights(x, wr, bias):
    logits = lax.dot_general(x, wr, (((1,), (1,)), ((), ())),
                             preferred_element_type=jnp.float32)
    scores = jax.nn.sigmoid(logits)
    biased = scores + bias
    ii = lax.broadcasted_iota(jnp.int32, biased.shape, 1)
    big = jnp.int32(NEXP)
    m1 = jnp.max(biased, axis=1, keepdims=True)
    i1 = jnp.min(jnp.where(biased >= m1, ii, big), axis=1, keepdims=True)
    oh1 = ii == i1
    b2 = jnp.where(oh1, -jnp.inf, biased)
    m2 = jnp.max(b2, axis=1, keepdims=True)
    i2 = jnp.min(jnp.where(b2 >= m2, ii, big), axis=1, keepdims=True)
    oh2 = ii == i2
    return jnp.where(oh1 | oh2, scores, 0.0)


def _router_body(x_ref, wr_ref, bias_ref, w_ref):
    w_ref[...] = _router_weights(x_ref[...], wr_ref[...], bias_ref[...])


def _router(x, wr, bias2d):
    return pl.pallas_call(
        _router_body,
        grid=(T // BT,),
        in_specs=[
            pl.BlockSpec((BT, D), lambda t: (t, 0)),
            pl.BlockSpec((NEXP, D), lambda t: (0, 0)),
            pl.BlockSpec((1, NEXP), lambda t: (0, 0)),
        ],
        out_specs=pl.BlockSpec((BT, NEXP), lambda t: (t, 0)),
        out_shape=jax.ShapeDtypeStruct((T, NEXP), jnp.float32),
    )(x, wr, bias2d)


def _dispatch_body(w_ref, posw_ref, meta_ref):
    w = w_ref[...]
    nslots = TOPK * T
    cap = _cap()
    nblk = _nblk()

    m = w > 0.0
    ii16 = lax.broadcasted_iota(jnp.int32, (T, NEXP), 1)
    e1 = jnp.min(jnp.where(m, ii16, NEXP + 1), axis=1, keepdims=True)
    e2 = jnp.max(jnp.where(m, ii16, -1), axis=1, keepdims=True)

    ii8 = lax.broadcasted_iota(jnp.int32, (T, E), 1)
    ind1 = ((ii8 == e1) & (e1 < E)).astype(jnp.float32)
    ind2 = ((ii8 == e2) & (e2 < E)).astype(jnp.float32)
    ind = jnp.concatenate([ind1, ind2], axis=0)

    chunk = 512 if nslots % 512 == 0 else nslots
    base = jnp.zeros((1, E), jnp.float32)
    r_chunks = []
    ir = lax.broadcasted_iota(jnp.int32, (chunk, chunk), 0)
    ic = lax.broadcasted_iota(jnp.int32, (chunk, chunk), 1)
    tri = (ic < ir).astype(jnp.float32)
    for c in range(nslots // chunk):
        ind_c = ind[c * chunk:(c + 1) * chunk]
        r_c = lax.dot_general(tri, ind_c, (((1,), (0,)), ((), ())),
                              preferred_element_type=jnp.float32)
        r_chunks.append(r_c + base)
        base = base + jnp.sum(ind_c, axis=0, keepdims=True)
    ranks = jnp.concatenate(r_chunks, axis=0)
    counts = base

    pc = jnp.floor((counts + (BROW - 1)) * (1.0 / BROW)) * BROW
    ie1 = lax.broadcasted_iota(jnp.int32, (E, E), 0)
    ie2 = lax.broadcasted_iota(jnp.int32, (E, E), 1)
    excl = (ie1 < ie2).astype(jnp.float32)
    starts = lax.dot_general(pc, excl, (((1,), (0,)), ((), ())),
                             preferred_element_type=jnp.float32)

    real = jnp.sum(ind, axis=1, keepdims=True) > 0.0
    pos_f = jnp.sum((ranks + starts) * ind, axis=1, keepdims=True)
    pos_f = jnp.where(real, pos_f, 0.0)

    slot_i = lax.broadcasted_iota(jnp.int32, (nslots, 1), 0).astype(jnp.float32)
    pos_s = jnp.where(real, pos_f, cap + slot_i)
    pos1 = pos_f[:T]
    pos2 = pos_f[T:]
    ps1 = pos_s[:T]
    ps2 = pos_s[T:]
    w1 = jnp.sum(w[:, :E] * ind1, axis=1, keepdims=True)
    w2 = jnp.sum(w[:, :E] * ind2, axis=1, keepdims=True)
    zero_w = jnp.sum(w[:, E:], axis=1, keepdims=True)
    pad = jnp.zeros((T, 128 - 7), jnp.float32)
    posw_ref[...] = jnp.concatenate(
        [pos1, pos2, w1, w2, zero_w, ps1, ps2, pad], axis=1)

    rowpos = (lax.broadcasted_iota(jnp.int32, (1, 128), 1) * BROW).astype(jnp.float32)
    be = jnp.zeros((1, 128), jnp.float32)
    for e in range(E):
        s_e = starts[:, e:e + 1]
        p_e = pc[:, e:e + 1]
        be = be + e * ((rowpos >= s_e) & (rowpos < s_e + p_e)).astype(jnp.float32)
    nused = jnp.sum(pc, axis=1, keepdims=True) * (1.0 / BROW)
    meta_ref[...] = jnp.concatenate([
        be.astype(jnp.int32),
        jnp.broadcast_to(nused.astype(jnp.int32), (1, 128)),
        jnp.zeros((6, 128), jnp.int32),
    ], axis=0)


def _dispatch(w_slots):
    return pl.pallas_call(
        _dispatch_body,
        grid=(1,),
        in_specs=[pl.BlockSpec((T, NEXP), lambda i: (0, 0))],
        out_specs=[
            pl.BlockSpec((T, 128), lambda i: (0, 0)),
            pl.BlockSpec((8, 128), lambda i: (0, 0)),
        ],
        out_shape=[
            jax.ShapeDtypeStruct((T, 128), jnp.float32),
            jax.ShapeDtypeStruct((8, 128), jnp.int32),
        ],
    )(w_slots)


def _sc_scatter_x(x, ps):
    n = ps.shape[0]
    wdt = x.shape[1]
    cap = _cap()
    info = plsc.get_sparse_core_info()
    nw = info.num_cores * info.num_subcores
    rows_w = n // nw
    mesh = plsc.VectorSubcoreMesh(core_axis_name="c", subcore_axis_name="s")

    chunk = rows_w
    while chunk * wdt * 4 > 280_000:
        chunk //= 2

    @functools.partial(
        pl.kernel, mesh=mesh,
        out_type=jax.ShapeDtypeStruct((cap + n, wdt), jnp.float32),
        scratch_types=[
            pltpu.VMEM((chunk,), jnp.int32),
            pltpu.VMEM((chunk, wdt), jnp.float32),
            pltpu.SemaphoreType.DMA,
        ],
    )
    def k(x_hbm, ps_hbm, xs_hbm, idx_v, rows_v, sem):
        wid = lax.axis_index("s") * info.num_cores + lax.axis_index("c")
        base = wid * rows_w
        for c in range(rows_w // chunk):
            tok0 = lax.rem(base + c * chunk, T)
            pltpu.sync_copy(ps_hbm.at[pl.ds(base + c * chunk, chunk)], idx_v)
            pltpu.sync_copy(x_hbm.at[pl.ds(tok0, chunk)], rows_v)
            pltpu.async_copy(rows_v, xs_hbm.at[idx_v], sem).wait()

    return k(x, ps)


def _sc_gather2(ys, idx1, idx2):
    n = idx1.shape[0]
    wdt = ys.shape[1]
    info = plsc.get_sparse_core_info()
    nw = info.num_cores * info.num_subcores
    rows_w = n // nw
    mesh = plsc.VectorSubcoreMesh(core_axis_name="c", subcore_axis_name="s")

    chunk = rows_w
    while chunk * wdt * 4 > 280_000:
        chunk //= 2

    @functools.partial(
        pl.kernel, mesh=mesh,
        out_type=(jax.ShapeDtypeStruct((n, wdt), jnp.float32),
                  jax.ShapeDtypeStruct((n, wdt), jnp.float32)),
        scratch_types=[
            pltpu.VMEM((chunk,), jnp.int32),
            pltpu.VMEM((chunk, wdt), jnp.float32),
            pltpu.SemaphoreType.DMA,
        ],
    )
    def k(ys_hbm, i1_hbm, i2_hbm, g1_hbm, g2_hbm, idx_v, rows_v, sem):
        wid = lax.axis_index("s") * info.num_cores + lax.axis_index("c")
        base = wid * rows_w
        for ih, oh in ((i1_hbm, g1_hbm), (i2_hbm, g2_hbm)):
            for c in range(rows_w // chunk):
                b = base + c * chunk
                pltpu.sync_copy(ih.at[pl.ds(b, chunk)], idx_v)
                pltpu.async_copy(ys_hbm.at[idx_v], rows_v, sem).wait()
                pltpu.sync_copy(rows_v, oh.at[pl.ds(b, chunk)])

    return k(ys, idx1, idx2)


def _ffn_body(be_ref, nu_ref, xs_ref, wg_ref, wu_ref, wd_ref, ys_ref):
    b = pl.program_id(0)

    @pl.when((b == 0) | (b < nu_ref[0]))
    def _go():
        x = xs_ref[...].astype(jnp.bfloat16)
        g = lax.dot_general(x, wg_ref[0], (((1,), (1,)), ((), ())),
                            preferred_element_type=jnp.float32)
        u = lax.dot_general(x, wu_ref[0], (((1,), (1,)), ((), ())),
                            preferred_element_type=jnp.float32)
        h = (g * jax.nn.sigmoid(g) * u).astype(jnp.bfloat16)
        ys_ref[...] = lax.dot_general(h, wd_ref[0], (((1,), (1,)), ((), ())),
                                      preferred_element_type=jnp.float32)


def _ffn(be, nu, xs, wgb, wub, wdb):
    cap = _cap()
    nblk = _nblk()
    grid_spec = pltpu.PrefetchScalarGridSpec(
        num_scalar_prefetch=2,
        grid=(nblk,),
        in_specs=[
            pl.BlockSpec((BROW, D), lambda b, be, nu: (jnp.minimum(b, jnp.maximum(nu[0] - 1, 0)), 0)),
            pl.BlockSpec((1, DFF, D), lambda b, be, nu: (be[b], 0, 0)),
            pl.BlockSpec((1, DFF, D), lambda b, be, nu: (be[b], 0, 0)),
            pl.BlockSpec((1, D, DFF), lambda b, be, nu: (be[b], 0, 0)),
        ],
        out_specs=pl.BlockSpec((BROW, D), lambda b, be, nu: (b, 0)),
    )
    return pl.pallas_call(
        _ffn_body,
        grid_spec=grid_spec,
        out_shape=jax.ShapeDtypeStruct((cap, D), jnp.float32),
        compiler_params=pltpu.CompilerParams(
            dimension_semantics=("arbitrary",),
        ),
    )(be, nu, xs, wgb, wub, wdb)


def _combine_body(posw_ref, x_ref, g1_ref, g2_ref, out_ref):
    pw = posw_ref[...]
    w1 = pw[:, 2:3]
    w2 = pw[:, 3:4]
    zw = pw[:, 4:5]
    out_ref[...] = (zw * x_ref[...]
                    + SCALE * (w1 * g1_ref[...] + w2 * g2_ref[...]))


def _combine(posw, x, g1, g2):
    return pl.pallas_call(
        _combine_body,
        grid=(T // BT,),
        in_specs=[
            pl.BlockSpec((BT, 128), lambda t: (t, 0)),
            pl.BlockSpec((BT, D), lambda t: (t, 0)),
            pl.BlockSpec((BT, D), lambda t: (t, 0)),
            pl.BlockSpec((BT, D), lambda t: (t, 0)),
        ],
        out_specs=pl.BlockSpec((BT, D), lambda t: (t, 0)),
        out_shape=jax.ShapeDtypeStruct((T, D), jnp.float32),
    )(posw, x, g1, g2)


@jax.jit
def _moe(x, wr, bias2d, wg, wu, wd):
    w_slots = _router(x, wr, bias2d)
    posw, meta = _dispatch(w_slots)
    be = meta[0, :_nblk()]
    nu = meta[1, :1]

    ps = jnp.concatenate([posw[:, 5], posw[:, 6]]).astype(jnp.int32)
    xs_full = _sc_scatter_x(x, ps)

    wgb = wg.astype(jnp.bfloat16)
    wub = wu.astype(jnp.bfloat16)
    wdb = wd.astype(jnp.bfloat16)
    ys = _ffn(be, nu, xs_full, wgb, wub, wdb)

    pos1 = posw[:, 0].astype(jnp.int32)
    pos2 = posw[:, 1].astype(jnp.int32)
    g1, g2 = _sc_gather2(ys, pos1, pos2)
    return _combine(posw, x, g1, g2)


def kernel(hidden_states, W_router, correction_bias, W_gate, W_up, W_down):
    bias2d = correction_bias.reshape(1, NEXP)
    return _moe(hidden_states, W_router, bias2d, W_gate, W_up, W_down)
